# baseline (speedup 1.0000x reference)
import functools

import jax
import jax.numpy as jnp
from jax import lax
from jax.experimental import pallas as pl
from jax.experimental.pallas import tpu as pltpu
from jax.experimental.pallas import tpu_sc as plsc

N_NODES = 10000
N_EDGES = 320000
D = 128

NC = 2
NS = 16
NW = NC * NS

BATCH = 128
NB = 80
CH = 8
NCH = NB // CH
EPT = NB * BATCH
E_PAD = EPT * NW
N_PAD = N_NODES + 8
ROWS_PER_SUB = 624
CHUNK = 104
NCHUNK = ROWS_PER_SUB // CHUNK
REM_BASE = ROWS_PER_SUB * NS
REM_ROWS = N_NODES - REM_BASE


def _sc_aggregate(x, src_t, dst_t):
  mesh = plsc.VectorSubcoreMesh(core_axis_name="c", subcore_axis_name="s")

  @functools.partial(
      pl.kernel,
      mesh=mesh,
      out_type=jax.ShapeDtypeStruct((NC, N_NODES, D), jnp.float32),
      scratch_types=[
          pltpu.VMEM((2, CH, BATCH), jnp.int32),
          pltpu.VMEM((2, CH, BATCH), jnp.int32),
          pltpu.VMEM((2, BATCH, D), jnp.float32),
          pltpu.VMEM_SHARED((N_PAD, D), jnp.float32),
          pltpu.SemaphoreType.DMA,
          pltpu.SemaphoreType.DMA,
          pltpu.SemaphoreType.DMA,
      ],
  )
  def k(x_hbm, src_hbm, dst_hbm, out_hbm, sidx, didx, rows, aggr,
        sem0, sem1, semi):
    c = lax.axis_index("c")
    s = lax.axis_index("s")
    wid = s * NC + c
    zbuf = rows.at[0]

    zero = jnp.zeros((16,), jnp.float32)

    def zrow(r, carry):
      for blk in range(D // 16):
        zbuf[r, pl.ds(blk * 16, 16)] = zero
      return carry

    lax.fori_loop(0, CHUNK, zrow, 0)
    r0 = s * ROWS_PER_SUB
    for kk in range(NCHUNK):
      pltpu.sync_copy(zbuf.at[pl.ds(0, CHUNK)],
                      aggr.at[pl.ds(r0 + kk * CHUNK, CHUNK)])

    @pl.when(s == NS - 1)
    def _zero_rem():
      pltpu.sync_copy(zbuf.at[pl.ds(0, REM_ROWS)],
                      aggr.at[pl.ds(REM_BASE, REM_ROWS)])

    pltpu.sync_copy(src_hbm.at[wid].at[pl.ds(0, CH)], sidx.at[0])
    pltpu.sync_copy(dst_hbm.at[wid].at[pl.ds(0, CH)], didx.at[0])
    plsc.subcore_barrier()

    sems = (sem0, sem1)
    pltpu.async_copy(x_hbm.at[sidx.at[0].at[0]], rows.at[0], sem0)
    pltpu.async_copy(x_hbm.at[sidx.at[0].at[1]], rows.at[1], sem1)

    def chunk_body(ci, carry):
      cur = ci % 2
      nxt = 1 - cur

      @pl.when(ci + 1 < NCH)
      def _prefetch_idx():
        nc = (ci + 1) * CH
        pltpu.async_copy(src_hbm.at[wid].at[pl.ds(nc, CH)], sidx.at[nxt],
                         semi)
        pltpu.async_copy(dst_hbm.at[wid].at[pl.ds(nc, CH)], didx.at[nxt],
                         semi)

      for b in range(CH):
        slot = b % 2
        sem = sems[slot]
        pltpu.make_async_copy(x_hbm.at[sidx.at[cur].at[b]], rows.at[slot],
                              sem).wait()
        pltpu.sync_copy(rows.at[slot], aggr.at[didx.at[cur].at[b]], add=True)

        if b == CH - 2:
          @pl.when(ci + 1 < NCH)
          def _wait_idx():
            pltpu.make_async_copy(src_hbm.at[wid].at[pl.ds(0, CH)],
                                  sidx.at[nxt], semi).wait()
            pltpu.make_async_copy(dst_hbm.at[wid].at[pl.ds(0, CH)],
                                  didx.at[nxt], semi).wait()

        if b < CH - 2:
          pltpu.async_copy(x_hbm.at[sidx.at[cur].at[b + 2]], rows.at[slot],
                           sem)
        else:

          @pl.when(ci + 1 < NCH)
          def _gather_next_chunk():
            pltpu.async_copy(x_hbm.at[sidx.at[nxt].at[b - (CH - 2)]],
                             rows.at[slot], sem)

      return carry

    lax.fori_loop(0, NCH, chunk_body, 0)
    plsc.subcore_barrier()

    for kk in range(NCHUNK):
      pltpu.sync_copy(aggr.at[pl.ds(r0 + kk * CHUNK, CHUNK)],
                      zbuf.at[pl.ds(0, CHUNK)])
      pltpu.sync_copy(zbuf.at[pl.ds(0, CHUNK)],
                      out_hbm.at[c].at[pl.ds(r0 + kk * CHUNK, CHUNK)])

    @pl.when(s == NS - 1)
    def _write_rem():
      pltpu.sync_copy(aggr.at[pl.ds(REM_BASE, REM_ROWS)],
                      rows.at[1].at[pl.ds(0, REM_ROWS)])
      pltpu.sync_copy(rows.at[1].at[pl.ds(0, REM_ROWS)],
                      out_hbm.at[c].at[pl.ds(REM_BASE, REM_ROWS)])

  return k(x, src_t, dst_t)


def _mlp_body(relu_out, x_ref, p_ref, wa_ref, ba_ref, wb_ref, bb_ref, o_ref):
  h = x_ref[...] + p_ref[0] + p_ref[1]
  t = jnp.dot(h, wa_ref[...], preferred_element_type=jnp.float32)
  t = jnp.maximum(t + ba_ref[...], 0.0)
  y = jnp.dot(t, wb_ref[...], preferred_element_type=jnp.float32)
  y = y + bb_ref[...]
  if relu_out:
    y = jnp.maximum(y, 0.0)
  o_ref[...] = y


_ROWS = 1000


def _tc_mlp(x, parts, Wa, ba, Wb, bb, relu_out):
  return pl.pallas_call(
      functools.partial(_mlp_body, relu_out),
      grid=(N_NODES // _ROWS,),
      in_specs=[
          pl.BlockSpec((_ROWS, D), lambda i: (i, 0)),
          pl.BlockSpec((NC, _ROWS, D), lambda i: (0, i, 0)),
          pl.BlockSpec((D, D), lambda i: (0, 0)),
          pl.BlockSpec((1, D), lambda i: (0, 0)),
          pl.BlockSpec((D, D), lambda i: (0, 0)),
          pl.BlockSpec((1, D), lambda i: (0, 0)),
      ],
      out_specs=pl.BlockSpec((_ROWS, D), lambda i: (i, 0)),
      out_shape=jax.ShapeDtypeStruct((N_NODES, D), jnp.float32),
  )(x, parts, Wa, ba.reshape(1, D), Wb, bb.reshape(1, D))


def kernel(x, edge_index, W1, b1, W2, b2, W3, b3, W4, b4):
  src = edge_index[0].astype(jnp.int32)
  dst = edge_index[1].astype(jnp.int32)
  pad = E_PAD - N_EDGES
  src_t = jnp.concatenate([src, jnp.zeros((pad,), jnp.int32)]).reshape(
      NW, NB, BATCH)
  dst_t = jnp.concatenate([dst, jnp.full((pad,), N_NODES, jnp.int32)]).reshape(
      NW, NB, BATCH)

  p1 = _sc_aggregate(x, src_t, dst_t)
  h = _tc_mlp(x, p1, W1, b1, W2, b2, relu_out=True)
  p2 = _sc_aggregate(h, src_t, dst_t)
  return _tc_mlp(h, p2, W3, b3, W4, b4, relu_out=False)
